# Initial kernel scaffold; baseline (speedup 1.0000x reference)
import jax
import jax.numpy as jnp
from jax.experimental import pallas as pl


def kernel(input, weight):
    raise NotImplementedError("write your pallas kernel here")


# speedup vs baseline: 1.8966x; 1.8966x over previous
import functools

import jax
import jax.numpy as jnp
from jax import lax
from jax.experimental import pallas as pl
from jax.experimental.pallas import tpu as pltpu
from jax.experimental.pallas import tpu_sc as plsc

B = 4096
LB = 50
D = 64
DP = 128
NW = 32
BPW = B // NW
G = 8
NCH = BPW // G
NV = D // 16

NROWS = 1000000
TN = 16384


def _bag_kernel(idx_hbm, w_hbm, out_hbm, idxv, rows, outv, sem):
    cid = lax.axis_index("c")
    sid = lax.axis_index("s")
    wid = sid * 2 + cid
    base = wid * BPW

    def chunk_body(g, carry):
        off = (base + g * G) * LB
        pltpu.sync_copy(idx_hbm.at[pl.ds(off, G * LB)], idxv)
        pltpu.async_copy(w_hbm.at[idxv], rows, sem).wait()
        for b in range(G):
            def red(r, acc):
                row = b * LB + r
                return tuple(
                    acc[d] + rows[row, pl.ds(d * 16, 16)] for d in range(NV)
                )
            acc0 = tuple(jnp.zeros((16,), jnp.float32) for _ in range(NV))
            acc = lax.fori_loop(0, LB, red, acc0)
            for d in range(NV):
                outv[g * G + b, pl.ds(d * 16, 16)] = acc[d] * (1.0 / LB)
        return carry

    lax.fori_loop(0, NCH, chunk_body, 0)
    pltpu.sync_copy(outv, out_hbm.at[pl.ds(base, BPW)])


def _tx_body(wt_ref, o_ref):
    row = jax.lax.broadcasted_iota(jnp.int32, (D, D), 0)
    col = jax.lax.broadcasted_iota(jnp.int32, (D, D), 1)
    ident = jnp.where(row == col, 1.0, 0.0).astype(jnp.float32)
    xt = jax.lax.dot_general(
        wt_ref[...], ident, (((0,), (0,)), ((), ())),
        preferred_element_type=jnp.float32,
    )
    o_ref[:, :D] = xt
    o_ref[:, D:] = jnp.zeros((TN, DP - D), jnp.float32)


def _transpose(wT):
    nblk = pl.cdiv(NROWS, TN)
    return pl.pallas_call(
        _tx_body,
        grid=(nblk,),
        in_specs=[pl.BlockSpec((D, TN), lambda j: (0, j))],
        out_specs=pl.BlockSpec((TN, DP), lambda j: (j, 0)),
        out_shape=jax.ShapeDtypeStruct((NROWS, DP), jnp.float32),
    )(wT)


def kernel(input, weight):
    idx_flat = input.reshape(-1)
    w2 = _transpose(weight.T)
    mesh = plsc.VectorSubcoreMesh(core_axis_name="c", subcore_axis_name="s")
    run = functools.partial(
        pl.kernel,
        mesh=mesh,
        out_type=jax.ShapeDtypeStruct((B, D), jnp.float32),
        scratch_types=[
            pltpu.VMEM((G * LB,), jnp.int32),
            pltpu.VMEM((G * LB, DP), jnp.float32),
            pltpu.VMEM((BPW, D), jnp.float32),
            pltpu.SemaphoreType.DMA,
        ],
    )(_bag_kernel)
    return run(idx_flat, w2)


# speedup vs baseline: 2.1795x; 1.1491x over previous
import functools

import jax
import jax.numpy as jnp
from jax import lax
from jax.experimental import pallas as pl
from jax.experimental.pallas import tpu as pltpu
from jax.experimental.pallas import tpu_sc as plsc

B = 4096
LB = 50
D = 64
NW = 32
BPW = B // NW
G = 8
NCH = BPW // G
NV = D // 16

NROWS = 1000000
TN = 32768
TN2 = TN // 2
NBLK = (NROWS + TN - 1) // TN
PROWS = NBLK * TN2


def _bag_kernel(idx_hbm, w_hbm, out_hbm, idxv, prow0, prow1, strip0, strip1,
                rows0, rows1, outv, sem0, sem1):
    cid = lax.axis_index("c")
    sid = lax.axis_index("s")
    wid = sid * 2 + cid
    base = wid * BPW

    prows = [prow0, prow1]
    strips = [strip0, strip1]
    rowbufs = [rows0, rows1]
    sems = [sem0, sem1]

    def prep(g):
        i = g % 2
        off = (base + g * G) * LB
        pltpu.sync_copy(idx_hbm.at[pl.ds(off, G * LB)], idxv)
        for k in range(G * LB // 16):
            vi = idxv[pl.ds(k * 16, 16)]
            blk = vi >> 15
            local = vi & (TN - 1)
            prows[i][pl.ds(k * 16, 16)] = (blk << 14) | (local & (TN2 - 1))
            strips[i][pl.ds(k * 16, 16)] = (local >> 14) * D
        return pltpu.async_copy(w_hbm.at[prows[i]], rowbufs[i], sems[i])

    handles = {0: prep(0)}
    for g in range(NCH):
        if g + 1 < NCH:
            handles[g + 1] = prep(g + 1)
        handles[g].wait()
        rows = rowbufs[g % 2]
        stripv = strips[g % 2]

        def bag_body(b, _):
            def red(r, acc):
                new = acc
                for rr in range(2):
                    row = b * LB + 2 * r + rr
                    sv = stripv[pl.ds(row, 16)]
                    cb = sv[0]
                    new = tuple(
                        new[d] + rows[row, pl.ds(cb + d * 16, 16)]
                        for d in range(NV)
                    )
                return new

            acc0 = tuple(jnp.zeros((16,), jnp.float32) for _ in range(NV))
            acc = lax.fori_loop(0, LB // 2, red, acc0)
            for d in range(NV):
                outv[g * G + b, pl.ds(d * 16, 16)] = acc[d] * (1.0 / LB)
            return _

        lax.fori_loop(0, G, bag_body, 0)
    pltpu.sync_copy(outv, out_hbm.at[pl.ds(base, BPW)])


def _tx_body(wt_ref, o_ref):
    row = jax.lax.broadcasted_iota(jnp.int32, (D, D), 0)
    col = jax.lax.broadcasted_iota(jnp.int32, (D, D), 1)
    ident = jnp.where(row == col, 1.0, 0.0).astype(jnp.float32)
    xt = jax.lax.dot_general(
        wt_ref[...], ident, (((0,), (0,)), ((), ())),
        preferred_element_type=jnp.float32,
    )
    o_ref[:, :D] = xt[:TN2, :]
    o_ref[:, D:] = xt[TN2:, :]


def _transpose(wT):
    return pl.pallas_call(
        _tx_body,
        grid=(NBLK,),
        in_specs=[pl.BlockSpec((D, TN), lambda j: (0, j))],
        out_specs=pl.BlockSpec((TN2, 128), lambda j: (j, 0)),
        out_shape=jax.ShapeDtypeStruct((PROWS, 128), jnp.float32),
    )(wT)


def kernel(input, weight):
    idx_flat = input.reshape(-1)
    w2 = _transpose(weight.T)
    mesh = plsc.VectorSubcoreMesh(core_axis_name="c", subcore_axis_name="s")
    run = functools.partial(
        pl.kernel,
        mesh=mesh,
        out_type=jax.ShapeDtypeStruct((B, D), jnp.float32),
        scratch_types=[
            pltpu.VMEM((G * LB,), jnp.int32),
            pltpu.VMEM((G * LB,), jnp.int32),
            pltpu.VMEM((G * LB,), jnp.int32),
            pltpu.VMEM((G * LB + 16,), jnp.int32),
            pltpu.VMEM((G * LB + 16,), jnp.int32),
            pltpu.VMEM((G * LB, 128), jnp.float32),
            pltpu.VMEM((G * LB, 128), jnp.float32),
            pltpu.VMEM((BPW, D), jnp.float32),
            pltpu.SemaphoreType.DMA,
            pltpu.SemaphoreType.DMA,
        ],
    )(_bag_kernel)
    return run(idx_flat, w2)
